# baseline (speedup 1.0000x reference)
import functools

import jax
import jax.numpy as jnp
from jax.experimental import pallas as pl
from jax.experimental.pallas import tpu as pltpu

F = 128


def _fused_body(nseg, ks, scale, *refs):
    a_refs = refs[0:nseg]
    x_refs = refs[nseg:2 * nseg]
    w_refs = refs[2 * nseg:3 * nseg]
    b_refs = refs[3 * nseg:4 * nseg]
    out_ref = refs[4 * nseg]
    h_ref = refs[4 * nseg + 1]

    m = pl.program_id(0)

    @pl.when(m == 0)
    def _():
        koff = 0
        for s in range(nseg):
            xb = x_refs[s][...]
            xc = jnp.concatenate([xb, xb * xb], axis=1)
            ht = jax.lax.dot_general(
                w_refs[s][...], xc, (((1,), (1,)), ((), ())),
                preferred_element_type=jnp.float32)
            h_ref[:, pl.ds(koff, ks[s])] = ht + b_refs[s][...]
            koff += ks[s]

    acc = None
    koff = 0
    for s in range(nseg):
        part = jax.lax.dot_general(
            h_ref[:, pl.ds(koff, ks[s])], a_refs[s][...],
            (((1,), (1,)), ((), ())),
            preferred_element_type=jnp.float32)
        acc = part if acc is None else acc + part
        koff += ks[s]

    y = scale * jnp.maximum(acc, 0.0)
    out_ref[...] = y.T


def _fused_output(a_list, x_list, w_list, b_list, scale, bm=256):
    nseg = len(a_list)
    m_rows = a_list[0].shape[0]
    ks = tuple(a.shape[1] for a in a_list)
    num_m = m_rows // bm

    b2_list = [b.reshape(F, 1) for b in b_list]

    a_specs = [pl.BlockSpec((bm, k), lambda mi: (mi, 0)) for k in ks]
    whole = lambda shape: pl.BlockSpec(shape, lambda mi: (0,) * len(shape))
    x_specs = [whole(x.shape) for x in x_list]
    w_specs = [whole(w.shape) for w in w_list]
    b_specs = [whole(b2.shape) for b2 in b2_list]
    out_spec = pl.BlockSpec((bm, F), lambda mi: (mi, 0))

    body = functools.partial(_fused_body, nseg, ks, scale)
    return pl.pallas_call(
        body,
        grid=(num_m,),
        in_specs=a_specs + x_specs + w_specs + b_specs,
        out_specs=out_spec,
        out_shape=jax.ShapeDtypeStruct((m_rows, F), jnp.float32),
        scratch_shapes=[pltpu.VMEM((F, sum(ks)), jnp.float32)],
        compiler_params=pltpu.CompilerParams(
            dimension_semantics=("arbitrary",)),
    )(*a_list, *x_list, *w_list, *b2_list)


def _manual_body(nseg, ks, num_m, bm, slots_t, scale, *refs):
    a_refs = refs[0:nseg]
    x_refs = refs[nseg:2 * nseg]
    w_refs = refs[2 * nseg:3 * nseg]
    b_refs = refs[3 * nseg:4 * nseg]
    out_ref = refs[4 * nseg]
    bufs = refs[4 * nseg + 1:5 * nseg + 1]
    h_ref = refs[5 * nseg + 1]
    ostage = refs[5 * nseg + 2]
    in_sems = refs[5 * nseg + 3]
    out_sems = refs[5 * nseg + 4]

    def in_copy(s, p, slot):
        return pltpu.make_async_copy(
            a_refs[s].at[pl.ds(p * bm, bm), :],
            bufs[s].at[slot],
            in_sems.at[s, slot])

    def out_copy(p):
        oslot = jax.lax.rem(p, 2)
        return pltpu.make_async_copy(
            ostage.at[oslot],
            out_ref.at[pl.ds(p * bm, bm), :],
            out_sems.at[oslot])

    for s in range(nseg):
        for q in range(slots_t[s] - 1):
            in_copy(s, q, q).start()

    koff = 0
    for s in range(nseg):
        for c in range(0, ks[s], 2048):
            cw = min(2048, ks[s] - c)
            xb = x_refs[s][pl.ds(c, cw), :]
            xc = jnp.concatenate([xb, xb * xb], axis=1)
            ht = jax.lax.dot_general(
                w_refs[s][...], xc, (((1,), (1,)), ((), ())),
                preferred_element_type=jnp.float32)
            h_ref[:, pl.ds(koff + c, cw)] = ht + b_refs[s][...]
        koff += ks[s]

    def step(p, carry):
        acc = None
        koff = 0
        for s in range(nseg):
            slot = jax.lax.rem(p, slots_t[s])
            in_copy(s, p, slot).wait()
            part = jax.lax.dot_general(
                h_ref[:, pl.ds(koff, ks[s])], bufs[s][slot],
                (((1,), (1,)), ((), ())),
                preferred_element_type=jnp.float32)
            acc = part if acc is None else acc + part
            koff += ks[s]

        for s in range(nseg):
            nxt = p + slots_t[s] - 1

            @pl.when(nxt < num_m)
            def _(s=s, nxt=nxt):
                in_copy(s, nxt, jax.lax.rem(nxt, slots_t[s])).start()

        y = scale * jnp.maximum(acc, 0.0)

        @pl.when(p >= 2)
        def _():
            out_copy(p - 2).wait()
        ostage[jax.lax.rem(p, 2)] = y.T
        out_copy(p).start()
        return carry

    jax.lax.fori_loop(0, num_m, step, 0)
    out_copy(num_m - 2).wait()
    out_copy(num_m - 1).wait()


def _fused_output_manual(a_list, x_list, w_list, b_list, scale, bm=256,
                         slots=None):
    nseg = len(a_list)
    m_rows = a_list[0].shape[0]
    ks = tuple(a.shape[1] for a in a_list)
    num_m = m_rows // bm
    if slots is None:
        slots = tuple(2 for k in ks)

    b2_list = [b.reshape(F, 1) for b in b_list]

    any_spec = pl.BlockSpec(memory_space=pltpu.MemorySpace.HBM)
    vmem_spec = pl.BlockSpec(memory_space=pltpu.MemorySpace.VMEM)

    body = functools.partial(_manual_body, nseg, ks, num_m, bm, slots, scale)
    return pl.pallas_call(
        body,
        in_specs=[any_spec] * nseg + [vmem_spec] * (3 * nseg),
        out_specs=any_spec,
        out_shape=jax.ShapeDtypeStruct((m_rows, F), jnp.float32),
        scratch_shapes=(
            [pltpu.VMEM((slots[s], bm, k), jnp.float32)
             for s, k in enumerate(ks)]
            + [pltpu.VMEM((F, sum(ks)), jnp.float32),
               pltpu.VMEM((2, bm, F), jnp.float32),
               pltpu.SemaphoreType.DMA((nseg, max(slots))),
               pltpu.SemaphoreType.DMA((2,))]),
        compiler_params=pltpu.CompilerParams(
            vmem_limit_bytes=61440 * 1000),
    )(*a_list, *x_list, *w_list, *b2_list)


def kernel(L0, L1, L2, D1invB1, D2B1TD1inv, B2TD2inv, B2D3, X0, X1, X2,
           Wn2n, bn2n, Wn2e, bn2e, We2e, be2e, We2n, be2n, We2t, be2t,
           Wt2e, bt2e, Wt2t, bt2t):
    Y0 = _fused_output_manual([L0, D1invB1], [X0, X1], [Wn2n, We2n], [bn2n, be2n],
                       0.5)
    Y1 = _fused_output_manual([L1, D2B1TD1inv, B2D3], [X1, X0, X2],
                       [We2e, Wn2e, Wt2e], [be2e, bn2e, bt2e], 1.0 / 3.0)
    Y2 = _fused_output_manual([L2, B2TD2inv], [X2, X1], [Wt2t, We2t], [bt2t, be2t],
                       0.5)
    return (Y0, Y1, Y2)


# speedup vs baseline: 1.5020x; 1.5020x over previous
import functools

import jax
import jax.numpy as jnp
from jax.experimental import pallas as pl
from jax.experimental.pallas import tpu as pltpu

F = 128


def _fused_body(nseg, ks, scale, *refs):
    a_refs = refs[0:nseg]
    x_refs = refs[nseg:2 * nseg]
    w_refs = refs[2 * nseg:3 * nseg]
    b_refs = refs[3 * nseg:4 * nseg]
    out_ref = refs[4 * nseg]
    h_ref = refs[4 * nseg + 1]

    m = pl.program_id(0)

    @pl.when(m == 0)
    def _():
        koff = 0
        for s in range(nseg):
            xb = x_refs[s][...]
            xc = jnp.concatenate([xb, xb * xb], axis=1)
            ht = jax.lax.dot_general(
                w_refs[s][...], xc, (((1,), (1,)), ((), ())),
                preferred_element_type=jnp.float32)
            h_ref[:, pl.ds(koff, ks[s])] = ht + b_refs[s][...]
            koff += ks[s]

    acc = None
    koff = 0
    for s in range(nseg):
        part = jax.lax.dot_general(
            h_ref[:, pl.ds(koff, ks[s])], a_refs[s][...],
            (((1,), (1,)), ((), ())),
            preferred_element_type=jnp.float32)
        acc = part if acc is None else acc + part
        koff += ks[s]

    y = scale * jnp.maximum(acc, 0.0)
    out_ref[...] = y.T


def _fused_output(a_list, x_list, w_list, b_list, scale, bm=256):
    nseg = len(a_list)
    m_rows = a_list[0].shape[0]
    ks = tuple(a.shape[1] for a in a_list)
    num_m = m_rows // bm

    b2_list = [b.reshape(F, 1) for b in b_list]

    a_specs = [pl.BlockSpec((bm, k), lambda mi: (mi, 0)) for k in ks]
    whole = lambda shape: pl.BlockSpec(shape, lambda mi: (0,) * len(shape))
    x_specs = [whole(x.shape) for x in x_list]
    w_specs = [whole(w.shape) for w in w_list]
    b_specs = [whole(b2.shape) for b2 in b2_list]
    out_spec = pl.BlockSpec((bm, F), lambda mi: (mi, 0))

    body = functools.partial(_fused_body, nseg, ks, scale)
    return pl.pallas_call(
        body,
        grid=(num_m,),
        in_specs=a_specs + x_specs + w_specs + b_specs,
        out_specs=out_spec,
        out_shape=jax.ShapeDtypeStruct((m_rows, F), jnp.float32),
        scratch_shapes=[pltpu.VMEM((F, sum(ks)), jnp.float32)],
        compiler_params=pltpu.CompilerParams(
            dimension_semantics=("arbitrary",)),
    )(*a_list, *x_list, *w_list, *b2_list)


def kernel(L0, L1, L2, D1invB1, D2B1TD1inv, B2TD2inv, B2D3, X0, X1, X2,
           Wn2n, bn2n, Wn2e, bn2e, We2e, be2e, We2n, be2n, We2t, be2t,
           Wt2e, bt2e, Wt2t, bt2t):
    Y0 = _fused_output([L0, D1invB1], [X0, X1], [Wn2n, We2n], [bn2n, be2n],
                       0.5)
    Y1 = _fused_output([L1, D2B1TD1inv, B2D3], [X1, X0, X2],
                       [We2e, Wn2e, Wt2e], [be2e, bn2e, bt2e], 1.0 / 3.0)
    Y2 = _fused_output([L2, B2TD2inv], [X2, X1], [Wt2t, We2t], [bt2t, be2t],
                       0.5)
    return (Y0, Y1, Y2)
